# Initial kernel scaffold; baseline (speedup 1.0000x reference)
import jax
import jax.numpy as jnp
from jax.experimental import pallas as pl


def kernel(input, adj, W, b):
    raise NotImplementedError("write your pallas kernel here")


# speedup vs baseline: 1.0059x; 1.0059x over previous
import jax
import jax.numpy as jnp
from jax.experimental import pallas as pl
from jax.experimental.pallas import tpu as pltpu

_TM = 400


def _fused_gcn_kernel(adj_ref, x_ref, w_ref, b_ref, o_ref):
    agg = jnp.dot(adj_ref[...], x_ref[...], preferred_element_type=jnp.float32)
    y = jax.lax.dot_general(
        agg, w_ref[...], (((1,), (1,)), ((), ())),
        preferred_element_type=jnp.float32,
    )
    o_ref[...] = jnp.tanh(y + b_ref[...])


def kernel(input, adj, W, b):
    n, k = adj.shape
    _, d = input.shape
    b2 = b.reshape(1, d)
    grid = (n // _TM,)
    return pl.pallas_call(
        _fused_gcn_kernel,
        grid=grid,
        in_specs=[
            pl.BlockSpec((_TM, k), lambda i: (i, 0)),
            pl.BlockSpec((k, d), lambda i: (0, 0)),
            pl.BlockSpec((d, d), lambda i: (0, 0)),
            pl.BlockSpec((1, d), lambda i: (0, 0)),
        ],
        out_specs=pl.BlockSpec((_TM, d), lambda i: (i, 0)),
        out_shape=jax.ShapeDtypeStruct((n, d), jnp.float32),
        compiler_params=pltpu.CompilerParams(
            dimension_semantics=("arbitrary",),
        ),
    )(adj, input, W, b2)
